# baseline (speedup 1.0000x reference)
import functools

import jax
import jax.numpy as jnp
from jax.experimental import pallas as pl
from jax.experimental.pallas import tpu as pltpu

B, N, IN_DIM, OUT_DIM, NUM_TYPES = 2, 4096, 64, 64, 4
BM = 512


def _hetero_iso_kernel(types_ref, feat_ref, w_f_ref, b_f_ref, w_g_ref,
                       b_g_ref, g_ref, out_ref, support_ref):
    i = pl.program_id(1)

    @pl.when(i == 0)
    def _compute_support():
        f = feat_ref[0]
        t = types_ref[0]
        w_g = w_g_ref[:]
        acc = jnp.zeros((N, OUT_DIM), jnp.float32)
        for ty in range(NUM_TYPES):
            wc = jnp.dot(w_f_ref[ty], w_g,
                         preferred_element_type=jnp.float32)
            bc = jnp.dot(b_f_ref[ty][None, :], w_g,
                         preferred_element_type=jnp.float32)
            mask = (t == ty).astype(jnp.float32)
            acc += jnp.dot(f * mask, wc,
                           preferred_element_type=jnp.float32)
            acc += mask * bc
        support_ref[:] = acc.astype(jnp.bfloat16)

    gblk = g_ref[0].astype(jnp.bfloat16)
    h = jnp.dot(gblk, support_ref[:], preferred_element_type=jnp.float32)
    h = h + b_g_ref[:]
    out_ref[0] = jnp.maximum(h, 0.0)


@jax.jit
def kernel(g, types, features, W_f, b_f, W_g, b_g):
    types_col = jnp.swapaxes(types, 1, 2).astype(jnp.int32)
    b_g2 = b_g.reshape(1, OUT_DIM)

    grid = (B, N // BM)
    out = pl.pallas_call(
        _hetero_iso_kernel,
        grid=grid,
        in_specs=[
            pl.BlockSpec((1, N, 1), lambda b, i: (b, 0, 0)),
            pl.BlockSpec((1, N, IN_DIM), lambda b, i: (b, 0, 0)),
            pl.BlockSpec((NUM_TYPES, IN_DIM, OUT_DIM),
                         lambda b, i: (0, 0, 0)),
            pl.BlockSpec((NUM_TYPES, OUT_DIM), lambda b, i: (0, 0)),
            pl.BlockSpec((IN_DIM, OUT_DIM), lambda b, i: (0, 0)),
            pl.BlockSpec((1, OUT_DIM), lambda b, i: (0, 0)),
            pl.BlockSpec((1, BM, N), lambda b, i: (b, i, 0)),
        ],
        out_specs=pl.BlockSpec((1, BM, OUT_DIM), lambda b, i: (b, i, 0)),
        out_shape=jax.ShapeDtypeStruct((B, N, OUT_DIM), jnp.float32),
        scratch_shapes=[pltpu.VMEM((N, OUT_DIM), jnp.bfloat16)],
        compiler_params=pltpu.CompilerParams(
            dimension_semantics=("arbitrary", "arbitrary"),
        ),
    )(types_col, features, W_f, b_f, W_g, b_g2, g)
    return out


# speedup vs baseline: 1.0024x; 1.0024x over previous
import functools

import jax
import jax.numpy as jnp
from jax.experimental import pallas as pl
from jax.experimental.pallas import tpu as pltpu

B, N, IN_DIM, OUT_DIM, NUM_TYPES = 2, 4096, 64, 64, 4
BM = 1024


def _hetero_iso_kernel(types_ref, feat_ref, w_f_ref, b_f_ref, w_g_ref,
                       b_g_ref, g_ref, out_ref, support_ref):
    i = pl.program_id(1)

    @pl.when(i == 0)
    def _compute_support():
        f = feat_ref[0]
        t = types_ref[0]
        w_g = w_g_ref[:]
        acc = jnp.zeros((N, OUT_DIM), jnp.float32)
        for ty in range(NUM_TYPES):
            wc = jnp.dot(w_f_ref[ty], w_g,
                         preferred_element_type=jnp.float32)
            bc = jnp.dot(b_f_ref[ty][None, :], w_g,
                         preferred_element_type=jnp.float32)
            mask = (t == ty).astype(jnp.float32)
            acc += jnp.dot(f * mask, wc,
                           preferred_element_type=jnp.float32)
            acc += mask * bc
        support_ref[:] = acc.astype(jnp.bfloat16)

    gblk = g_ref[0].astype(jnp.bfloat16)
    h = jnp.dot(gblk, support_ref[:], preferred_element_type=jnp.float32)
    h = h + b_g_ref[:]
    out_ref[0] = jnp.maximum(h, 0.0)


@jax.jit
def kernel(g, types, features, W_f, b_f, W_g, b_g):
    types_col = jnp.swapaxes(types, 1, 2).astype(jnp.int32)
    b_g2 = b_g.reshape(1, OUT_DIM)

    grid = (B, N // BM)
    out = pl.pallas_call(
        _hetero_iso_kernel,
        grid=grid,
        in_specs=[
            pl.BlockSpec((1, N, 1), lambda b, i: (b, 0, 0)),
            pl.BlockSpec((1, N, IN_DIM), lambda b, i: (b, 0, 0)),
            pl.BlockSpec((NUM_TYPES, IN_DIM, OUT_DIM),
                         lambda b, i: (0, 0, 0)),
            pl.BlockSpec((NUM_TYPES, OUT_DIM), lambda b, i: (0, 0)),
            pl.BlockSpec((IN_DIM, OUT_DIM), lambda b, i: (0, 0)),
            pl.BlockSpec((1, OUT_DIM), lambda b, i: (0, 0)),
            pl.BlockSpec((1, BM, N), lambda b, i: (b, i, 0)),
        ],
        out_specs=pl.BlockSpec((1, BM, OUT_DIM), lambda b, i: (b, i, 0)),
        out_shape=jax.ShapeDtypeStruct((B, N, OUT_DIM), jnp.float32),
        scratch_shapes=[pltpu.VMEM((N, OUT_DIM), jnp.bfloat16)],
        compiler_params=pltpu.CompilerParams(
            dimension_semantics=("arbitrary", "arbitrary"),
        ),
    )(types_col, features, W_f, b_f, W_g, b_g2, g)
    return out
